# baseline (speedup 1.0000x reference)
import jax
import jax.numpy as jnp
from jax.experimental import pallas as pl
from jax.experimental.pallas import tpu as pltpu

_H = 24
_W = 24
_P = _H * _W
_N = _P + 1
_TOP_N = 3
_T = 84
_DIST = 280.0
_NUM_LAYERS = 9
_D = 1024
_HEADS = 16
_NEG = -1e30
_B = 4
_K = 12


def _finish_batch(b, attnbuf, acc_ref, hsbuf, agg_ref, bench_ref):
    f32 = jnp.float32
    hi = jax.lax.Precision.HIGHEST
    r_i = jax.lax.broadcasted_iota(jnp.int32, (1, _N), 1)
    m_raw = jnp.sum(attnbuf[b, :, 0, :], axis=0, keepdims=True)
    m_row = jnp.where(r_i == 0, f32(_NEG), m_raw)

    q_i = jax.lax.broadcasted_iota(jnp.int32, (_N, _N), 0)
    p_i = jax.lax.broadcasted_iota(jnp.int32, (_N, _N), 1)
    eye = jnp.where(q_i == p_i, f32(1.0), f32(0.0))
    m_col = jax.lax.dot_general(
        eye, m_row, (((1,), (1,)), ((), ())),
        precision=hi, preferred_element_type=f32)

    rid_row = ((r_i - 1) // (2 * _W)) * (_W // 2) + ((r_i - 1) % _W) // 2
    c_i = jax.lax.broadcasted_iota(jnp.int32, (_N, 1), 0)
    rid_col = ((c_i - 1) // (2 * _W)) * (_W // 2) + ((c_i - 1) % _W) // 2
    same_region = rid_col == rid_row

    gt_qp = (m_col > m_row) | ((m_col == m_row) & (q_i < p_i))
    gt_pq = (m_row > m_col) | ((m_row == m_col) & (p_i < q_i))

    one, zero = jnp.int32(1), jnp.int32(0)

    def count_rows(mask):
        return jnp.sum(jnp.where(mask, one, zero), axis=0, keepdims=True)

    def count_cols(mask):
        return jnp.sum(jnp.where(mask, one, zero), axis=1, keepdims=True)

    cand_row = (count_rows(same_region & gt_qp) < _TOP_N) & (r_i > 0)
    cand_col = (count_cols(same_region & gt_pq) < _TOP_N) & \
        (jax.lax.broadcasted_iota(jnp.int32, (_N, 1), 0) > 0)

    sel_row = cand_row & (count_rows(cand_col & gt_qp) < _T)
    sel_col = cand_col & (count_cols(cand_row & gt_pq) < _T)

    pos_row = count_rows(sel_col & (q_i < p_i))
    pos_col = count_cols(sel_row & (p_i < q_i))

    t_i = jax.lax.broadcasted_iota(jnp.int32, (_T, _N), 0)
    j_i = jax.lax.broadcasted_iota(jnp.int32, (_T, _N), 1)
    oh = sel_row & (pos_row == t_i)
    tT_i = jax.lax.broadcasted_iota(jnp.int32, (_N, _T), 1)
    pT_i = jax.lax.broadcasted_iota(jnp.int32, (_N, _T), 0)
    ohT = sel_col & (pos_col == tT_i)

    bench_row = jnp.sum(jnp.where(ohT, pT_i - 1, zero), axis=0,
                        keepdims=True)
    bench_col = jnp.sum(jnp.where(oh, j_i - 1, zero), axis=1,
                        keepdims=True)
    bench_scores = jnp.sum(jnp.where(oh, m_row, f32(0.0)), axis=1,
                           keepdims=True)

    lt_qp = (~gt_qp) & (q_i != p_i)
    srank = count_rows(sel_col & lt_qp)
    lo_idx = int(0.55 * (_T - 1))
    v_lo = jnp.sum(jnp.where(sel_row & (srank == lo_idx), m_row,
                             f32(0.0)), axis=1, keepdims=True)
    v_hi = jnp.sum(jnp.where(sel_row & (srank == lo_idx + 1), m_row,
                             f32(0.0)), axis=1, keepdims=True)
    frac = f32(0.55) * f32(_T - 1) - f32(lo_idx)
    thr = v_lo + frac * (v_hi - v_lo)
    is_high = bench_scores >= thr[0, 0]

    pt = acc_ref[...] * f32(1.0 / _NUM_LAYERS)
    norm = jnp.sqrt(jnp.sum(pt * pt, axis=1, keepdims=True))
    ptn = pt / jnp.maximum(norm, f32(1e-12))

    ohf = jnp.where(oh, f32(1.0), f32(0.0))
    df = jax.lax.Precision.DEFAULT
    q_rows = jax.lax.dot_general(
        ohf, ptn, (((1,), (0,)), ((), ())),
        precision=df, preferred_element_type=f32)
    sim_b = jax.lax.dot_general(
        q_rows, ptn, (((1,), (1,)), ((), ())),
        precision=df, preferred_element_type=f32)

    yt = (bench_col // _W).astype(f32)
    xt = (bench_col % _W).astype(f32)
    yj = ((j_i - 1) // _W).astype(f32)
    xj = ((j_i - 1) % _W).astype(f32)
    dy = yt - yj
    dx = xt - xj
    dist = jnp.sqrt(dy * dy + dx * dx)
    dp = f32(1.0) - jnp.minimum(dist * f32(1.0 / (_DIST ** 0.5)),
                                f32(1.0))

    bw = jnp.maximum(sim_b, f32(0.0)) * dp
    sel_m = jnp.where(is_high, f32(0.0), f32(1.0))
    sel_m = jnp.where(oh, f32(1.0), sel_m)
    sel_m = jnp.where(j_i == 0, f32(0.0), sel_m)
    bw = bw * sel_m
    den = jnp.sum(bw, axis=1, keepdims=True) + f32(1e-8)
    bwn = bw / den
    bwn = jnp.where(oh, f32(1.0), bwn)

    agg = jax.lax.dot_general(
        bwn, hsbuf[b], (((1,), (0,)), ((), ())),
        precision=df, preferred_element_type=f32)
    agg_ref[b] = agg
    bench_ref[b] = bench_row.astype(jnp.int32)


def _body(attn_hbm, st_hbm, hs_hbm, agg_ref, bench_ref,
          ring, acc_ref, hsbuf, attnbuf, ring_sem, hs_sem, attn_sem):
    total = _B * _NUM_LAYERS
    copies = []
    for g in range(total):
        b, l = divmod(g, _NUM_LAYERS)
        copies.append(pltpu.make_async_copy(
            st_hbm.at[l, b], ring.at[g % _K], ring_sem.at[g % _K]))
    hs_cps = [pltpu.make_async_copy(hs_hbm.at[b], hsbuf.at[b], hs_sem.at[b])
              for b in range(_B)]
    at_cps = [pltpu.make_async_copy(attn_hbm.at[b, :, 0:1, :],
                                    attnbuf.at[b], attn_sem.at[b])
              for b in range(_B)]
    for c in at_cps:
        c.start()
    for c in hs_cps:
        c.start()
    for g in range(_K):
        copies[g].start()
    for gg in range(total // 3):
        b, lg = divmod(gg, _NUM_LAYERS // 3)
        g0 = gg * 3
        for g in range(g0, g0 + 3):
            copies[g].wait()
        grp = (ring[g0 % _K] + ring[(g0 + 1) % _K]) + ring[(g0 + 2) % _K]
        if lg == 0:
            acc_ref[...] = grp
        else:
            acc_ref[...] = acc_ref[...] + grp
        for g in range(g0, g0 + 3):
            if g + _K < total:
                copies[g + _K].start()
        if lg == _NUM_LAYERS // 3 - 1:
            at_cps[b].wait()
            hs_cps[b].wait()
            _finish_batch(b, attnbuf, acc_ref, hsbuf, agg_ref, bench_ref)


@jax.jit
def kernel(hidden_states_sel, stacked_hs, attn):
    B = hidden_states_sel.shape[0]
    agg, bench = pl.pallas_call(
        _body,
        in_specs=[
            pl.BlockSpec(memory_space=pl.ANY),
            pl.BlockSpec(memory_space=pl.ANY),
            pl.BlockSpec(memory_space=pl.ANY),
        ],
        out_shape=[
            jax.ShapeDtypeStruct((B, _T, _D), jnp.float32),
            jax.ShapeDtypeStruct((B, 1, _T), jnp.int32),
        ],
        scratch_shapes=[
            pltpu.VMEM((_K, _N, _D), jnp.float32),
            pltpu.VMEM((_N, _D), jnp.float32),
            pltpu.VMEM((_B, _N, _D), jnp.float32),
            pltpu.VMEM((_B, _HEADS, 1, _N), jnp.float32),
            pltpu.SemaphoreType.DMA((_K,)),
            pltpu.SemaphoreType.DMA((_B,)),
            pltpu.SemaphoreType.DMA((_B,)),
        ],
    )(attn, stacked_hs, hidden_states_sel)
    return agg, bench.reshape(B, _T)


# speedup vs baseline: 1.0078x; 1.0078x over previous
import jax
import jax.numpy as jnp
from jax.experimental import pallas as pl
from jax.experimental.pallas import tpu as pltpu

_H = 24
_W = 24
_P = _H * _W
_N = _P + 1
_TOP_N = 3
_T = 84
_DIST = 280.0
_NUM_LAYERS = 9
_D = 1024
_HEADS = 16
_NEG = -1e30
_B = 4
_K = 9


def _finish_batch(b, attnbuf, acc_ref, hsbuf, agg_ref, bench_ref):
    f32 = jnp.float32
    hi = jax.lax.Precision.HIGHEST
    r_i = jax.lax.broadcasted_iota(jnp.int32, (1, _N), 1)
    m_raw = jnp.sum(attnbuf[b, :, 0, :], axis=0, keepdims=True)
    m_row = jnp.where(r_i == 0, f32(_NEG), m_raw)

    q_i = jax.lax.broadcasted_iota(jnp.int32, (_N, _N), 0)
    p_i = jax.lax.broadcasted_iota(jnp.int32, (_N, _N), 1)
    eye = jnp.where(q_i == p_i, f32(1.0), f32(0.0))
    m_col = jax.lax.dot_general(
        eye, m_row, (((1,), (1,)), ((), ())),
        precision=hi, preferred_element_type=f32)

    rid_row = ((r_i - 1) // (2 * _W)) * (_W // 2) + ((r_i - 1) % _W) // 2
    c_i = jax.lax.broadcasted_iota(jnp.int32, (_N, 1), 0)
    rid_col = ((c_i - 1) // (2 * _W)) * (_W // 2) + ((c_i - 1) % _W) // 2
    same_region = rid_col == rid_row

    gt_qp = (m_col > m_row) | ((m_col == m_row) & (q_i < p_i))
    gt_pq = (m_row > m_col) | ((m_row == m_col) & (p_i < q_i))

    one, zero = jnp.int32(1), jnp.int32(0)

    def count_rows(mask):
        return jnp.sum(jnp.where(mask, one, zero), axis=0, keepdims=True)

    def count_cols(mask):
        return jnp.sum(jnp.where(mask, one, zero), axis=1, keepdims=True)

    cand_row = (count_rows(same_region & gt_qp) < _TOP_N) & (r_i > 0)
    cand_col = (count_cols(same_region & gt_pq) < _TOP_N) & \
        (jax.lax.broadcasted_iota(jnp.int32, (_N, 1), 0) > 0)

    sel_row = cand_row & (count_rows(cand_col & gt_qp) < _T)
    sel_col = cand_col & (count_cols(cand_row & gt_pq) < _T)

    pos_row = count_rows(sel_col & (q_i < p_i))
    pos_col = count_cols(sel_row & (p_i < q_i))

    t_i = jax.lax.broadcasted_iota(jnp.int32, (_T, _N), 0)
    j_i = jax.lax.broadcasted_iota(jnp.int32, (_T, _N), 1)
    oh = sel_row & (pos_row == t_i)
    tT_i = jax.lax.broadcasted_iota(jnp.int32, (_N, _T), 1)
    pT_i = jax.lax.broadcasted_iota(jnp.int32, (_N, _T), 0)
    ohT = sel_col & (pos_col == tT_i)

    bench_row = jnp.sum(jnp.where(ohT, pT_i - 1, zero), axis=0,
                        keepdims=True)
    bench_col = jnp.sum(jnp.where(oh, j_i - 1, zero), axis=1,
                        keepdims=True)
    bench_scores = jnp.sum(jnp.where(oh, m_row, f32(0.0)), axis=1,
                           keepdims=True)

    lt_qp = (~gt_qp) & (q_i != p_i)
    srank = count_rows(sel_col & lt_qp)
    lo_idx = int(0.55 * (_T - 1))
    v_lo = jnp.sum(jnp.where(sel_row & (srank == lo_idx), m_row,
                             f32(0.0)), axis=1, keepdims=True)
    v_hi = jnp.sum(jnp.where(sel_row & (srank == lo_idx + 1), m_row,
                             f32(0.0)), axis=1, keepdims=True)
    frac = f32(0.55) * f32(_T - 1) - f32(lo_idx)
    thr = v_lo + frac * (v_hi - v_lo)
    is_high = bench_scores >= thr[0, 0]

    pt = acc_ref[...] * f32(1.0 / _NUM_LAYERS)
    norm = jnp.sqrt(jnp.sum(pt * pt, axis=1, keepdims=True))
    ptn = pt / jnp.maximum(norm, f32(1e-12))

    ohf = jnp.where(oh, f32(1.0), f32(0.0))
    df = jax.lax.Precision.DEFAULT
    q_rows = jax.lax.dot_general(
        ohf, ptn, (((1,), (0,)), ((), ())),
        precision=df, preferred_element_type=f32)
    sim_b = jax.lax.dot_general(
        q_rows, ptn, (((1,), (1,)), ((), ())),
        precision=df, preferred_element_type=f32)

    yt = (bench_col // _W).astype(f32)
    xt = (bench_col % _W).astype(f32)
    yj = ((j_i - 1) // _W).astype(f32)
    xj = ((j_i - 1) % _W).astype(f32)
    dy = yt - yj
    dx = xt - xj
    dist = jnp.sqrt(dy * dy + dx * dx)
    dp = f32(1.0) - jnp.minimum(dist * f32(1.0 / (_DIST ** 0.5)),
                                f32(1.0))

    bw = jnp.maximum(sim_b, f32(0.0)) * dp
    sel_m = jnp.where(is_high, f32(0.0), f32(1.0))
    sel_m = jnp.where(oh, f32(1.0), sel_m)
    sel_m = jnp.where(j_i == 0, f32(0.0), sel_m)
    bw = bw * sel_m
    den = jnp.sum(bw, axis=1, keepdims=True) + f32(1e-8)
    bwn = bw / den
    bwn = jnp.where(oh, f32(1.0), bwn)

    agg = jax.lax.dot_general(
        bwn, hsbuf[b], (((1,), (0,)), ((), ())),
        precision=df, preferred_element_type=f32)
    agg_ref[b] = agg
    bench_ref[b] = bench_row.astype(jnp.int32)


class _Pair:
    def __init__(self, a, b):
        self._a, self._b = a, b

    def start(self):
        self._a.start()
        self._b.start()

    def wait(self):
        self._a.wait()
        self._b.wait()


def _body(attn_hbm, st_hbm, hs_hbm, agg_ref, bench_ref,
          ring, acc_ref, hsbuf, attnbuf, ring_sem, ring_semb, hs_sem,
          attn_sem):
    total = _B * _NUM_LAYERS
    hd = _D // 2
    copies = []
    for g in range(total):
        b, l = divmod(g, _NUM_LAYERS)
        s = g % _K
        copies.append(_Pair(
            pltpu.make_async_copy(st_hbm.at[l, b, :, 0:hd],
                                  ring.at[s, :, 0:hd], ring_sem.at[s]),
            pltpu.make_async_copy(st_hbm.at[l, b, :, hd:_D],
                                  ring.at[s, :, hd:_D], ring_semb.at[s])))
    hs_cps = [pltpu.make_async_copy(hs_hbm.at[b], hsbuf.at[b], hs_sem.at[b])
              for b in range(_B)]
    at_cps = [pltpu.make_async_copy(attn_hbm.at[b, :, 0:1, :],
                                    attnbuf.at[b], attn_sem.at[b])
              for b in range(_B)]
    for c in at_cps:
        c.start()
    for c in hs_cps:
        c.start()
    for g in range(_K):
        copies[g].start()
    for gg in range(total // 3):
        b, lg = divmod(gg, _NUM_LAYERS // 3)
        g0 = gg * 3
        for g in range(g0, g0 + 3):
            copies[g].wait()
        grp = (ring[g0 % _K] + ring[(g0 + 1) % _K]) + ring[(g0 + 2) % _K]
        if lg == 0:
            acc_ref[...] = grp
        else:
            acc_ref[...] = acc_ref[...] + grp
        for g in range(g0, g0 + 3):
            if g + _K < total:
                copies[g + _K].start()
        if lg == _NUM_LAYERS // 3 - 1:
            at_cps[b].wait()
            hs_cps[b].wait()
            _finish_batch(b, attnbuf, acc_ref, hsbuf, agg_ref, bench_ref)


@jax.jit
def kernel(hidden_states_sel, stacked_hs, attn):
    B = hidden_states_sel.shape[0]
    agg, bench = pl.pallas_call(
        _body,
        in_specs=[
            pl.BlockSpec(memory_space=pl.ANY),
            pl.BlockSpec(memory_space=pl.ANY),
            pl.BlockSpec(memory_space=pl.ANY),
        ],
        out_shape=[
            jax.ShapeDtypeStruct((B, _T, _D), jnp.float32),
            jax.ShapeDtypeStruct((B, 1, _T), jnp.int32),
        ],
        scratch_shapes=[
            pltpu.VMEM((_K, _N, _D), jnp.float32),
            pltpu.VMEM((_N, _D), jnp.float32),
            pltpu.VMEM((_B, _N, _D), jnp.float32),
            pltpu.VMEM((_B, _HEADS, 1, _N), jnp.float32),
            pltpu.SemaphoreType.DMA((_K,)),
            pltpu.SemaphoreType.DMA((_K,)),
            pltpu.SemaphoreType.DMA((_B,)),
            pltpu.SemaphoreType.DMA((_B,)),
        ],
    )(attn, stacked_hs, hidden_states_sel)
    return agg, bench.reshape(B, _T)
